# baseline (speedup 1.0000x reference)
import functools
import math

import jax
import jax.numpy as jnp
from jax import lax
from jax.experimental import pallas as pl
from jax.experimental.pallas import tpu as pltpu
from jax.experimental.pallas import tpu_sc as plsc

_VOCAB = 1000000
_HIDDEN = 64
_SCALE = math.sqrt(_HIDDEN)

_B, _T = 16384, 50
_LANES = 128
_NBT = _B // _LANES
_NW = 32
_BT_PER_W = _NBT // _NW
_NBUF = 5
_LEAD = 4

_mesh = plsc.VectorSubcoreMesh(core_axis_name="c", subcore_axis_name="s")


@functools.partial(
    pl.kernel,
    mesh=_mesh,
    out_type=jax.ShapeDtypeStruct((_T, 8, _NBT, 8, _LANES), jnp.float32),
    scratch_types=[
        pltpu.VMEM((_LANES, _T), jnp.int32),
        pltpu.VMEM((_NBUF, _LANES), jnp.int32),
        pltpu.VMEM((_NBUF * _LANES, _HIDDEN), jnp.float32),
        pltpu.VMEM((_NBUF, 8, 8, _LANES), jnp.float32),
        pltpu.SemaphoreType.DMA,
        pltpu.SemaphoreType.DMA,
        pltpu.SemaphoreType.DMA,
        pltpu.SemaphoreType.DMA,
        pltpu.SemaphoreType.DMA,
        pltpu.SemaphoreType.DMA,
        pltpu.SemaphoreType.DMA,
        pltpu.SemaphoreType.DMA,
        pltpu.SemaphoreType.DMA,
        pltpu.SemaphoreType.DMA,
    ],
    compiler_params=pltpu.CompilerParams(
        use_tc_tiling_on_sc=False, needs_layout_passes=False),
)
def _embed(ids_hbm, table_hbm, out_hbm, ids_v, idxb, rows_v, outb,
           g0, g1, g2, g3, g4, o0, o1, o2, o3, o4):
    gsems = (g0, g1, g2, g3, g4)
    osems = (o0, o1, o2, o3, o4)
    nc = 2
    wid = lax.axis_index("s") * nc + lax.axis_index("c")

    iota = lax.iota(jnp.int32, 16)
    base8 = [iota + 16 * dg for dg in range(8)]

    def build_idx(slot, t):
        ts = jnp.full((16,), t, jnp.int32)
        for g in range(8):
            v = plsc.load_gather(ids_v, [base8[g], ts])
            idxb[slot, pl.ds(16 * g, 16)] = v

    def rows_slice(slot):
        return rows_v.at[pl.ds(slot * _LANES, _LANES)]

    def fire_gather(slot):
        pltpu.async_copy(table_hbm.at[idxb.at[slot]], rows_slice(slot),
                         gsems[slot])

    def drain_gather(slot):
        pltpu.make_async_copy(table_hbm.at[idxb.at[slot]], rows_slice(slot),
                              gsems[slot]).wait()

    def transform(slot):
        rowg = [base8[dg] + slot * _LANES for dg in range(8)]

        def h_body(h, _):
            a = h // 8
            c = h % 8
            cs = jnp.full((16,), h, jnp.int32)
            for dg in range(8):
                v = plsc.load_gather(rows_v, [rowg[dg], cs])
                outb[slot, a, c, pl.ds(dg * 16, 16)] = v * _SCALE
            return 0

        lax.fori_loop(0, _HIDDEN, h_body, 0)

    def fire_write(slot, t, bt):
        pltpu.async_copy(outb.at[slot], out_hbm.at[t, :, bt], osems[slot])

    def drain_write(slot, t, bt):
        pltpu.make_async_copy(outb.at[slot], out_hbm.at[t, :, bt],
                              osems[slot]).wait()

    def bt_body(i, _):
        bt = wid * _BT_PER_W + i
        pltpu.sync_copy(ids_hbm.at[pl.ds(bt * _LANES, _LANES)], ids_v)
        for u in range(_LEAD):
            build_idx(u, u)
            fire_gather(u)

        def kk_body(kk, _):
            for s in range(_NBUF):
                u = kk * _NBUF + s
                drain_gather(s)
                pl.when(kk >= 1)(lambda s=s, u=u: drain_write(s, u, bt))
                transform(s)
                fire_write(s, u, bt)

                def refill(s=s, u=u):
                    s2 = (s + _LEAD) % _NBUF
                    build_idx(s2, u + _LEAD)
                    fire_gather(s2)

                pl.when(u + _LEAD < _T)(refill)
            return 0

        lax.fori_loop(0, _T // _NBUF, kk_body, 0)
        for s in range(_NBUF):
            drain_write(s, s, bt)
        return 0

    lax.fori_loop(0, _BT_PER_W, bt_body, 0)


def kernel(input_ids, embed_weight):
    ids = input_ids.astype(jnp.int32)
    out5 = _embed(ids, embed_weight)
    return jnp.transpose(out5, (2, 4, 0, 1, 3)).reshape(_B, _T, _HIDDEN)


# speedup vs baseline: 1.5361x; 1.5361x over previous
import functools
import math

import jax
import jax.numpy as jnp
from jax import lax
from jax.experimental import pallas as pl
from jax.experimental.pallas import tpu as pltpu
from jax.experimental.pallas import tpu_sc as plsc

_VOCAB = 1000000
_HIDDEN = 64
_SCALE = math.sqrt(_HIDDEN)

_B, _T = 16384, 50
_LANES = 128
_NBT = _B // _LANES
_NW = 32
_BT_PER_W = _NBT // _NW
_NBUF = 5
_LEAD = 4

_mesh = plsc.VectorSubcoreMesh(core_axis_name="c", subcore_axis_name="s")


@functools.partial(
    pl.kernel,
    mesh=_mesh,
    out_type=jax.ShapeDtypeStruct((_T, 8, _NBT, 8, _LANES), jnp.float32),
    scratch_types=[
        pltpu.VMEM((_LANES, _T), jnp.int32),
        pltpu.VMEM((_NBUF, _LANES), jnp.int32),
        pltpu.VMEM((_NBUF * _LANES, _HIDDEN), jnp.float32),
        pltpu.VMEM((_NBUF, 8, 8, _LANES), jnp.float32),
        pltpu.SemaphoreType.DMA,
        pltpu.SemaphoreType.DMA,
        pltpu.SemaphoreType.DMA,
        pltpu.SemaphoreType.DMA,
        pltpu.SemaphoreType.DMA,
        pltpu.SemaphoreType.DMA,
        pltpu.SemaphoreType.DMA,
        pltpu.SemaphoreType.DMA,
        pltpu.SemaphoreType.DMA,
        pltpu.SemaphoreType.DMA,
    ],
    compiler_params=pltpu.CompilerParams(
        use_tc_tiling_on_sc=False, needs_layout_passes=False),
)
def _embed(ids_hbm, table_hbm, out_hbm, ids_v, idxb, rows_v, outb,
           g0, g1, g2, g3, g4, o0, o1, o2, o3, o4):
    gsems = (g0, g1, g2, g3, g4)
    osems = (o0, o1, o2, o3, o4)
    nc = 2
    wid = lax.axis_index("s") * nc + lax.axis_index("c")

    iota = lax.iota(jnp.int32, 16)
    base8 = [iota + 16 * dg for dg in range(8)]

    def build_idx(slot, t):
        ts = jnp.full((16,), t, jnp.int32)
        for g in range(8):
            v = plsc.load_gather(ids_v, [base8[g], ts])
            idxb[slot, pl.ds(16 * g, 16)] = v

    def rows_slice(slot):
        return rows_v.at[pl.ds(slot * _LANES, _LANES)]

    def fire_gather(slot):
        pltpu.async_copy(table_hbm.at[idxb.at[slot]], rows_slice(slot),
                         gsems[slot])

    def drain_gather(slot):
        pltpu.make_async_copy(table_hbm.at[idxb.at[slot]], rows_slice(slot),
                              gsems[slot]).wait()

    def transform(slot):
        rowg = [base8[dg] + slot * _LANES for dg in range(8)]

        @plsc.parallel_loop(0, _HIDDEN, unroll=4)
        def h_body(h):
            a = h // 8
            c = h % 8
            cs = jnp.full((16,), h, jnp.int32)
            for dg in range(8):
                v = plsc.load_gather(rows_v, [rowg[dg], cs])
                outb[slot, a, c, pl.ds(dg * 16, 16)] = v * _SCALE

    def fire_write(slot, t, bt):
        pltpu.async_copy(outb.at[slot], out_hbm.at[t, :, bt], osems[slot])

    def drain_write(slot, t, bt):
        pltpu.make_async_copy(outb.at[slot], out_hbm.at[t, :, bt],
                              osems[slot]).wait()

    def bt_body(i, _):
        bt = wid * _BT_PER_W + i
        pltpu.sync_copy(ids_hbm.at[pl.ds(bt * _LANES, _LANES)], ids_v)
        for u in range(_LEAD):
            build_idx(u, u)
            fire_gather(u)

        def kk_body(kk, _):
            for s in range(_NBUF):
                u = kk * _NBUF + s
                drain_gather(s)
                pl.when(kk >= 1)(lambda s=s, u=u: drain_write(s, u, bt))
                transform(s)
                fire_write(s, u, bt)

                def refill(s=s, u=u):
                    s2 = (s + _LEAD) % _NBUF
                    build_idx(s2, u + _LEAD)
                    fire_gather(s2)

                pl.when(u + _LEAD < _T)(refill)
            return 0

        lax.fori_loop(0, _T // _NBUF, kk_body, 0)
        for s in range(_NBUF):
            drain_write(s, s, bt)
        return 0

    lax.fori_loop(0, _BT_PER_W, bt_body, 0)


def kernel(input_ids, embed_weight):
    ids = input_ids.astype(jnp.int32)
    out5 = _embed(ids, embed_weight)
    return jnp.transpose(out5, (2, 4, 0, 1, 3)).reshape(_B, _T, _HIDDEN)


# speedup vs baseline: 2.5555x; 1.6636x over previous
import functools
import math

import jax
import jax.numpy as jnp
from jax import lax
from jax.experimental import pallas as pl
from jax.experimental.pallas import tpu as pltpu
from jax.experimental.pallas import tpu_sc as plsc

_VOCAB = 1000000
_HIDDEN = 64
_SCALE = math.sqrt(_HIDDEN)

_B, _T = 16384, 50
_LANES = 128
_NBT = _B // _LANES
_NW = 32
_BT_PER_W = _NBT // _NW
_NBUF = 5
_LEAD = 4

_mesh = plsc.VectorSubcoreMesh(core_axis_name="c", subcore_axis_name="s")


@functools.partial(
    pl.kernel,
    mesh=_mesh,
    out_type=jax.ShapeDtypeStruct((_T, 8, _NBT, 8, _LANES), jnp.float32),
    scratch_types=[
        pltpu.VMEM((_LANES, _T), jnp.int32),
        pltpu.VMEM((_NBUF, _LANES), jnp.int32),
        pltpu.VMEM((_NBUF * _LANES, _HIDDEN), jnp.float32),
        pltpu.VMEM((_NBUF, 8, 8, _LANES), jnp.float32),
        pltpu.VMEM((_LANES * 65 + 16,), jnp.float32),
        pltpu.SemaphoreType.DMA,
        pltpu.SemaphoreType.DMA,
        pltpu.SemaphoreType.DMA,
        pltpu.SemaphoreType.DMA,
        pltpu.SemaphoreType.DMA,
        pltpu.SemaphoreType.DMA,
        pltpu.SemaphoreType.DMA,
        pltpu.SemaphoreType.DMA,
        pltpu.SemaphoreType.DMA,
        pltpu.SemaphoreType.DMA,
    ],
    compiler_params=pltpu.CompilerParams(
        use_tc_tiling_on_sc=False, needs_layout_passes=False),
)
def _embed(ids_hbm, table_hbm, out_hbm, ids_v, idxb, rows_v, outb, stg,
           g0, g1, g2, g3, g4, o0, o1, o2, o3, o4):
    gsems = (g0, g1, g2, g3, g4)
    osems = (o0, o1, o2, o3, o4)
    nc = 2
    wid = lax.axis_index("s") * nc + lax.axis_index("c")

    iota = lax.iota(jnp.int32, 16)
    base8 = [iota + 16 * dg for dg in range(8)]

    def build_idx(slot, t):
        ts = jnp.full((16,), t, jnp.int32)
        for g in range(8):
            v = plsc.load_gather(ids_v, [base8[g], ts])
            idxb[slot, pl.ds(16 * g, 16)] = v

    def rows_slice(slot):
        return rows_v.at[pl.ds(slot * _LANES, _LANES)]

    def fire_gather(slot):
        pltpu.async_copy(table_hbm.at[idxb.at[slot]], rows_slice(slot),
                         gsems[slot])

    def drain_gather(slot):
        pltpu.make_async_copy(table_hbm.at[idxb.at[slot]], rows_slice(slot),
                              gsems[slot]).wait()

    _P = 65
    flat65 = [(iota + dg * 16) * _P for dg in range(8)]

    def transform(slot):
        @plsc.parallel_loop(0, _LANES, unroll=2)
        def skew_body(r):
            row = slot * _LANES + r
            sb = r * _P
            for k in range(4):
                v = rows_v[row, pl.ds(16 * k, 16)]
                plsc.store_scatter(stg, [iota + (sb + 16 * k)], v)

        @plsc.parallel_loop(0, _HIDDEN, unroll=4)
        def h_body(h):
            a = h // 8
            c = h % 8
            for dg in range(8):
                v = plsc.load_gather(stg, [flat65[dg] + h])
                outb[slot, a, c, pl.ds(dg * 16, 16)] = v * _SCALE

    def fire_write(slot, t, bt):
        pltpu.async_copy(outb.at[slot], out_hbm.at[t, :, bt], osems[slot])

    def drain_write(slot, t, bt):
        pltpu.make_async_copy(outb.at[slot], out_hbm.at[t, :, bt],
                              osems[slot]).wait()

    def bt_body(i, _):
        bt = wid * _BT_PER_W + i
        pltpu.sync_copy(ids_hbm.at[pl.ds(bt * _LANES, _LANES)], ids_v)
        for u in range(_LEAD):
            build_idx(u, u)
            fire_gather(u)

        def kk_body(kk, _):
            for s in range(_NBUF):
                u = kk * _NBUF + s
                drain_gather(s)
                pl.when(kk >= 1)(lambda s=s, u=u: drain_write(s, u, bt))
                transform(s)
                fire_write(s, u, bt)

                def refill(s=s, u=u):
                    s2 = (s + _LEAD) % _NBUF
                    build_idx(s2, u + _LEAD)
                    fire_gather(s2)

                pl.when(u + _LEAD < _T)(refill)
            return 0

        lax.fori_loop(0, _T // _NBUF, kk_body, 0)
        for s in range(_NBUF):
            drain_write(s, s, bt)
        return 0

    lax.fori_loop(0, _BT_PER_W, bt_body, 0)


def kernel(input_ids, embed_weight):
    ids = input_ids.astype(jnp.int32)
    out5 = _embed(ids, embed_weight)
    return jnp.transpose(out5, (2, 4, 0, 1, 3)).reshape(_B, _T, _HIDDEN)


# speedup vs baseline: 4.6794x; 1.8311x over previous
import functools
import math

import jax
import jax.numpy as jnp
from jax import lax
from jax.experimental import pallas as pl
from jax.experimental.pallas import tpu as pltpu
from jax.experimental.pallas import tpu_sc as plsc

_VOCAB = 1000000
_HIDDEN = 64
_SCALE = math.sqrt(_HIDDEN)

_B, _T = 16384, 50
_LANES = 128
_NBT = _B // _LANES
_NW = 32
_BT_PER_W = _NBT // _NW
_NBUF = 5
_LEAD = 4

_mesh = plsc.VectorSubcoreMesh(core_axis_name="c", subcore_axis_name="s")

_VPAD = 1000064
_NVB = _VPAD // 128
_NKA = 246


@functools.partial(
    pl.kernel,
    mesh=_mesh,
    out_type=jax.ShapeDtypeStruct((_VPAD // 2, 128), jnp.float32),
    scratch_types=[
        pltpu.VMEM((2, _HIDDEN, 128), jnp.float32),
        pltpu.VMEM((2, _HIDDEN, 128), jnp.float32),
        pltpu.VMEM((128 * 65 + 16,), jnp.float32),
        pltpu.SemaphoreType.DMA,
        pltpu.SemaphoreType.DMA,
        pltpu.SemaphoreType.DMA,
        pltpu.SemaphoreType.DMA,
    ],
    compiler_params=pltpu.CompilerParams(
        use_tc_tiling_on_sc=True, needs_layout_passes=False),
)
def _compact(tt_hbm, ctab_hbm, inb, onb, stg, i0, i1, q0, q1):
    isems = (i0, i1)
    qsems = (q0, q1)
    nc = 2
    wid = lax.axis_index("s") * nc + lax.axis_index("c")

    iota = lax.iota(jnp.int32, 16)
    flat65 = [(iota + dg * 16) * 65 for dg in range(8)]

    def vblk(j):
        return j * _NW + wid

    def src_slice(j):
        return tt_hbm.at[:, pl.ds(vblk(j) * 128, 128)]

    def fire_in(j, s):
        pltpu.async_copy(src_slice(j), inb.at[s], isems[s])

    def drain_in(j, s):
        pltpu.make_async_copy(src_slice(j), inb.at[s], isems[s]).wait()

    def dst_slice(j):
        return ctab_hbm.at[pl.ds(vblk(j) * 64, 64)]

    def fire_out(j, s):
        pltpu.async_copy(onb.at[s], dst_slice(j), qsems[s])

    def drain_out(j, s):
        pltpu.make_async_copy(onb.at[s], dst_slice(j), qsems[s]).wait()

    def transform(s):
        @plsc.parallel_loop(0, _HIDDEN, unroll=2)
        def p0(h):
            for dg in range(8):
                v = inb[s, h, pl.ds(16 * dg, 16)]
                plsc.store_scatter(stg, [flat65[dg] + h], v)

        @plsc.parallel_loop(0, 128, unroll=2)
        def p1(d):
            d2 = d // 2
            off = (d % 2) * _HIDDEN
            base = d * 65
            for k in range(4):
                v = plsc.load_gather(stg, [iota + (base + 16 * k)])
                onb[s, d2, pl.ds(off + 16 * k, 16)] = v

    fire_in(0, 0)

    def m_body(m, _):
        for s in range(2):
            j = 2 * m + s

            def prefetch(j=j, s=s):
                fire_in(j + 1, (s + 1) % 2)

            pl.when((j + 1 < _NKA - 1) & (vblk(j + 1) < _NVB))(prefetch)

            def process(j=j, s=s):
                drain_in(j, s)
                pl.when(j >= 2)(lambda: drain_out(j - 2, s))
                transform(s)
                fire_out(j, s)

            pl.when((j < _NKA - 1) & (vblk(j) < _NVB))(process)
        return 0

    lax.fori_loop(0, _NKA // 2, m_body, 0)
    drain_out(_NKA - 3, 1)
    drain_out(_NKA - 4, 0)


@functools.partial(
    pl.kernel,
    mesh=_mesh,
    out_type=jax.ShapeDtypeStruct((_T, 8, _NBT, 8, _LANES), jnp.float32),
    scratch_types=[
        pltpu.VMEM((_LANES, _T), jnp.int32),
        pltpu.VMEM((_NBUF, _LANES), jnp.int32),
        pltpu.VMEM((_NBUF * _LANES, _HIDDEN), jnp.float32),
        pltpu.VMEM((_NBUF, 8, 8, _LANES), jnp.float32),
        pltpu.VMEM((_LANES * 65 + 16,), jnp.float32),
        pltpu.SemaphoreType.DMA,
        pltpu.SemaphoreType.DMA,
        pltpu.SemaphoreType.DMA,
        pltpu.SemaphoreType.DMA,
        pltpu.SemaphoreType.DMA,
        pltpu.SemaphoreType.DMA,
        pltpu.SemaphoreType.DMA,
        pltpu.SemaphoreType.DMA,
        pltpu.SemaphoreType.DMA,
        pltpu.SemaphoreType.DMA,
    ],
    compiler_params=pltpu.CompilerParams(
        use_tc_tiling_on_sc=False, needs_layout_passes=False),
)
def _embed(ids_hbm, table_hbm, out_hbm, ids_v, idxb, rows_v, outb, stg,
           g0, g1, g2, g3, g4, o0, o1, o2, o3, o4):
    gsems = (g0, g1, g2, g3, g4)
    osems = (o0, o1, o2, o3, o4)
    nc = 2
    wid = lax.axis_index("s") * nc + lax.axis_index("c")

    iota = lax.iota(jnp.int32, 16)
    base8 = [iota + 16 * dg for dg in range(8)]

    def build_idx(slot, t):
        ts = jnp.full((16,), t, jnp.int32)
        for g in range(8):
            v = plsc.load_gather(ids_v, [base8[g], ts])
            idxb[slot, pl.ds(16 * g, 16)] = v

    def rows_slice(slot):
        return rows_v.at[pl.ds(slot * _LANES, _LANES)]

    def fire_gather(slot):
        pltpu.async_copy(table_hbm.at[idxb.at[slot]], rows_slice(slot),
                         gsems[slot])

    def drain_gather(slot):
        pltpu.make_async_copy(table_hbm.at[idxb.at[slot]], rows_slice(slot),
                              gsems[slot]).wait()

    _P = 65
    flat65 = [(iota + dg * 16) * _P for dg in range(8)]

    def transform(slot):
        @plsc.parallel_loop(0, _LANES, unroll=2)
        def skew_body(r):
            row = slot * _LANES + r
            sb = r * _P
            for k in range(4):
                v = rows_v[row, pl.ds(16 * k, 16)]
                plsc.store_scatter(stg, [iota + (sb + 16 * k)], v)

        @plsc.parallel_loop(0, _HIDDEN, unroll=4)
        def h_body(h):
            a = h // 8
            c = h % 8
            for dg in range(8):
                v = plsc.load_gather(stg, [flat65[dg] + h])
                outb[slot, a, c, pl.ds(dg * 16, 16)] = v * _SCALE

    def fire_write(slot, t, bt):
        pltpu.async_copy(outb.at[slot], out_hbm.at[t, :, bt], osems[slot])

    def drain_write(slot, t, bt):
        pltpu.make_async_copy(outb.at[slot], out_hbm.at[t, :, bt],
                              osems[slot]).wait()

    def bt_body(i, _):
        bt = wid * _BT_PER_W + i
        pltpu.sync_copy(ids_hbm.at[pl.ds(bt * _LANES, _LANES)], ids_v)
        for u in range(_LEAD):
            build_idx(u, u)
            fire_gather(u)

        def kk_body(kk, _):
            for s in range(_NBUF):
                u = kk * _NBUF + s
                drain_gather(s)
                pl.when(kk >= 1)(lambda s=s, u=u: drain_write(s, u, bt))
                transform(s)
                fire_write(s, u, bt)

                def refill(s=s, u=u):
                    s2 = (s + _LEAD) % _NBUF
                    build_idx(s2, u + _LEAD)
                    fire_gather(s2)

                pl.when(u + _LEAD < _T)(refill)
            return 0

        lax.fori_loop(0, _T // _NBUF, kk_body, 0)
        for s in range(_NBUF):
            drain_write(s, s, bt)
        return 0

    lax.fori_loop(0, _BT_PER_W, bt_body, 0)


def kernel(input_ids, embed_weight):
    ids = input_ids.astype(jnp.int32)
    ctab = _compact(embed_weight.T)
    out5 = _embed(ids, ctab.reshape(_VPAD, _HIDDEN))
    return jnp.transpose(out5, (2, 4, 0, 1, 3)).reshape(_B, _T, _HIDDEN)


# speedup vs baseline: 5.3224x; 1.1374x over previous
import functools
import math

import jax
import jax.numpy as jnp
from jax import lax
from jax.experimental import pallas as pl
from jax.experimental.pallas import tpu as pltpu
from jax.experimental.pallas import tpu_sc as plsc

_VOCAB = 1000000
_HIDDEN = 64
_SCALE = math.sqrt(_HIDDEN)

_B, _T = 16384, 50
_LANES = 128
_NBT = _B // _LANES
_NW = 32
_BT_PER_W = _NBT // _NW
_NBUF = 5
_LEAD = 4

_mesh = plsc.VectorSubcoreMesh(core_axis_name="c", subcore_axis_name="s")

_VPAD = 1000064
_NVB = _VPAD // 128
_NKA = 246


@functools.partial(
    pl.kernel,
    mesh=_mesh,
    out_type=jax.ShapeDtypeStruct((_VPAD // 2, 128), jnp.float32),
    scratch_types=[
        pltpu.VMEM((3, _HIDDEN, 128), jnp.float32),
        pltpu.VMEM((3, _HIDDEN, 128), jnp.float32),
        pltpu.VMEM((128 * 65 + 16,), jnp.float32),
        pltpu.SemaphoreType.DMA,
        pltpu.SemaphoreType.DMA,
        pltpu.SemaphoreType.DMA,
        pltpu.SemaphoreType.DMA,
        pltpu.SemaphoreType.DMA,
        pltpu.SemaphoreType.DMA,
    ],
    compiler_params=pltpu.CompilerParams(
        use_tc_tiling_on_sc=True, needs_layout_passes=False),
)
def _compact(tt_hbm, ctab_hbm, inb, onb, stg, i0, i1, i2, q0, q1, q2):
    isems = (i0, i1, i2)
    qsems = (q0, q1, q2)
    nc = 2
    wid = lax.axis_index("s") * nc + lax.axis_index("c")

    iota = lax.iota(jnp.int32, 16)
    flat65 = [(iota + dg * 16) * 65 for dg in range(8)]

    def vblk(j):
        return j * _NW + wid

    def src_slice(j):
        return tt_hbm.at[:, pl.ds(vblk(j) * 128, 128)]

    def fire_in(j, s):
        pltpu.async_copy(src_slice(j), inb.at[s], isems[s])

    def drain_in(j, s):
        pltpu.make_async_copy(src_slice(j), inb.at[s], isems[s]).wait()

    def dst_slice(j):
        return ctab_hbm.at[pl.ds(vblk(j) * 64, 64)]

    def fire_out(j, s):
        pltpu.async_copy(onb.at[s], dst_slice(j), qsems[s])

    def drain_out(j, s):
        pltpu.make_async_copy(onb.at[s], dst_slice(j), qsems[s]).wait()

    def transform(s):
        @plsc.parallel_loop(0, _HIDDEN, unroll=2)
        def p0(h):
            for dg in range(8):
                v = inb[s, h, pl.ds(16 * dg, 16)]
                plsc.store_scatter(stg, [flat65[dg] + h], v)

        @plsc.parallel_loop(0, 128, unroll=2)
        def p1(d):
            d2 = d // 2
            off = (d % 2) * _HIDDEN
            base = d * 65
            for k in range(4):
                v = plsc.load_gather(stg, [iota + (base + 16 * k)])
                onb[s, d2, pl.ds(off + 16 * k, 16)] = v

    fire_in(0, 0)
    fire_in(1, 1)

    def m_body(m, _):
        for s in range(3):
            j = 3 * m + s

            def prefetch(j=j, s=s):
                fire_in(j + 2, (s + 2) % 3)

            pl.when((j + 2 < _NKA - 1) & (vblk(j + 2) < _NVB))(prefetch)

            def process(j=j, s=s):
                drain_in(j, s)
                pl.when(j >= 3)(lambda: drain_out(j - 3, s))
                transform(s)
                fire_out(j, s)

            pl.when((j < _NKA - 1) & (vblk(j) < _NVB))(process)
        return 0

    lax.fori_loop(0, _NKA // 3, m_body, 0)
    drain_out(_NKA - 3, 0)
    drain_out(_NKA - 5, 1)
    drain_out(_NKA - 4, 2)


@functools.partial(
    pl.kernel,
    mesh=_mesh,
    out_type=jax.ShapeDtypeStruct((_T, 8, _NBT, 8, _LANES), jnp.float32),
    scratch_types=[
        pltpu.VMEM((_LANES, _T), jnp.int32),
        pltpu.VMEM((_NBUF, _LANES), jnp.int32),
        pltpu.VMEM((_NBUF * _LANES, _HIDDEN), jnp.float32),
        pltpu.VMEM((_NBUF, 8, 8, _LANES), jnp.float32),
        pltpu.VMEM((_LANES * 65 + 16,), jnp.float32),
        pltpu.SemaphoreType.DMA,
        pltpu.SemaphoreType.DMA,
        pltpu.SemaphoreType.DMA,
        pltpu.SemaphoreType.DMA,
        pltpu.SemaphoreType.DMA,
        pltpu.SemaphoreType.DMA,
        pltpu.SemaphoreType.DMA,
        pltpu.SemaphoreType.DMA,
        pltpu.SemaphoreType.DMA,
        pltpu.SemaphoreType.DMA,
    ],
    compiler_params=pltpu.CompilerParams(
        use_tc_tiling_on_sc=False, needs_layout_passes=False),
)
def _embed(ids_hbm, table_hbm, out_hbm, ids_v, idxb, rows_v, outb, stg,
           g0, g1, g2, g3, g4, o0, o1, o2, o3, o4):
    gsems = (g0, g1, g2, g3, g4)
    osems = (o0, o1, o2, o3, o4)
    nc = 2
    wid = lax.axis_index("s") * nc + lax.axis_index("c")

    iota = lax.iota(jnp.int32, 16)
    base8 = [iota + 16 * dg for dg in range(8)]

    def build_idx(slot, t):
        ts = jnp.full((16,), t, jnp.int32)
        for g in range(8):
            v = plsc.load_gather(ids_v, [base8[g], ts])
            idxb[slot, pl.ds(16 * g, 16)] = v

    def rows_slice(slot):
        return rows_v.at[pl.ds(slot * _LANES, _LANES)]

    def fire_gather(slot):
        pltpu.async_copy(table_hbm.at[idxb.at[slot]], rows_slice(slot),
                         gsems[slot])

    def drain_gather(slot):
        pltpu.make_async_copy(table_hbm.at[idxb.at[slot]], rows_slice(slot),
                              gsems[slot]).wait()

    _P = 65
    flat65 = [(iota + dg * 16) * _P for dg in range(8)]

    def transform(slot):
        @plsc.parallel_loop(0, _LANES, unroll=2)
        def skew_body(r):
            row = slot * _LANES + r
            sb = r * _P
            for k in range(4):
                v = rows_v[row, pl.ds(16 * k, 16)]
                plsc.store_scatter(stg, [iota + (sb + 16 * k)], v)

        @plsc.parallel_loop(0, _HIDDEN, unroll=4)
        def h_body(h):
            a = h // 8
            c = h % 8
            for dg in range(8):
                v = plsc.load_gather(stg, [flat65[dg] + h])
                outb[slot, a, c, pl.ds(dg * 16, 16)] = v * _SCALE

    def fire_write(slot, t, bt):
        pltpu.async_copy(outb.at[slot], out_hbm.at[t, :, bt], osems[slot])

    def drain_write(slot, t, bt):
        pltpu.make_async_copy(outb.at[slot], out_hbm.at[t, :, bt],
                              osems[slot]).wait()

    def bt_body(i, _):
        bt = wid * _BT_PER_W + i
        pltpu.sync_copy(ids_hbm.at[pl.ds(bt * _LANES, _LANES)], ids_v)
        for u in range(_LEAD):
            build_idx(u, u)
            fire_gather(u)

        def kk_body(kk, _):
            for s in range(_NBUF):
                u = kk * _NBUF + s
                drain_gather(s)
                pl.when(kk >= 1)(lambda s=s, u=u: drain_write(s, u, bt))
                transform(s)
                fire_write(s, u, bt)

                def refill(s=s, u=u):
                    s2 = (s + _LEAD) % _NBUF
                    build_idx(s2, u + _LEAD)
                    fire_gather(s2)

                pl.when(u + _LEAD < _T)(refill)
            return 0

        lax.fori_loop(0, _T // _NBUF, kk_body, 0)
        for s in range(_NBUF):
            drain_write(s, s, bt)
        return 0

    lax.fori_loop(0, _BT_PER_W, bt_body, 0)


def kernel(input_ids, embed_weight):
    ids = input_ids.astype(jnp.int32)
    ctab = _compact(embed_weight.T)
    out5 = _embed(ids, ctab.reshape(_VPAD, _HIDDEN))
    return jnp.transpose(out5, (2, 4, 0, 1, 3)).reshape(_B, _T, _HIDDEN)


# speedup vs baseline: 5.4007x; 1.0147x over previous
import functools
import math

import jax
import jax.numpy as jnp
from jax import lax
from jax.experimental import pallas as pl
from jax.experimental.pallas import tpu as pltpu
from jax.experimental.pallas import tpu_sc as plsc

_VOCAB = 1000000
_HIDDEN = 64
_SCALE = math.sqrt(_HIDDEN)

_B, _T = 16384, 50
_LANES = 128
_NBT = _B // _LANES
_NW = 32
_BT_PER_W = _NBT // _NW
_NBUF = 5
_LEAD = 4

_mesh = plsc.VectorSubcoreMesh(core_axis_name="c", subcore_axis_name="s")

_VPAD = 1000064
_NVB = _VPAD // 128
_NKA = 246


@functools.partial(
    pl.kernel,
    mesh=_mesh,
    out_type=jax.ShapeDtypeStruct((_VPAD // 2, 128), jnp.float32),
    scratch_types=[
        pltpu.VMEM((4, _HIDDEN, 128), jnp.float32),
        pltpu.VMEM((4, _HIDDEN, 128), jnp.float32),
        pltpu.VMEM((128 * 65 + 16,), jnp.float32),
        pltpu.SemaphoreType.DMA,
        pltpu.SemaphoreType.DMA,
        pltpu.SemaphoreType.DMA,
        pltpu.SemaphoreType.DMA,
        pltpu.SemaphoreType.DMA,
        pltpu.SemaphoreType.DMA,
        pltpu.SemaphoreType.DMA,
        pltpu.SemaphoreType.DMA,
    ],
    compiler_params=pltpu.CompilerParams(
        use_tc_tiling_on_sc=True, needs_layout_passes=False),
)
def _compact(tt_hbm, ctab_hbm, inb, onb, stg,
             i0, i1, i2, i3, q0, q1, q2, q3):
    isems = (i0, i1, i2, i3)
    qsems = (q0, q1, q2, q3)
    nc = 2
    wid = lax.axis_index("s") * nc + lax.axis_index("c")

    iota = lax.iota(jnp.int32, 16)
    flat65 = [(iota + dg * 16) * 65 for dg in range(8)]

    def vblk(j):
        return j * _NW + wid

    def src_slice(j):
        return tt_hbm.at[:, pl.ds(vblk(j) * 128, 128)]

    def fire_in(j, s):
        pltpu.async_copy(src_slice(j), inb.at[s], isems[s])

    def drain_in(j, s):
        pltpu.make_async_copy(src_slice(j), inb.at[s], isems[s]).wait()

    def dst_slice(j):
        return ctab_hbm.at[pl.ds(vblk(j) * 64, 64)]

    def fire_out(j, s):
        pltpu.async_copy(onb.at[s], dst_slice(j), qsems[s])

    def drain_out(j, s):
        pltpu.make_async_copy(onb.at[s], dst_slice(j), qsems[s]).wait()

    def transform(s):
        @plsc.parallel_loop(0, _HIDDEN, unroll=2)
        def p0(h):
            for dg in range(8):
                v = inb[s, h, pl.ds(16 * dg, 16)]
                plsc.store_scatter(stg, [flat65[dg] + h], v)

        @plsc.parallel_loop(0, 128, unroll=2)
        def p1(d):
            d2 = d // 2
            off = (d % 2) * _HIDDEN
            base = d * 65
            for k in range(4):
                v = plsc.load_gather(stg, [iota + (base + 16 * k)])
                onb[s, d2, pl.ds(off + 16 * k, 16)] = v

    fire_in(0, 0)
    fire_in(1, 1)
    fire_in(2, 2)

    def m_body(m, _):
        for s in range(4):
            j = 4 * m + s

            def prefetch(j=j, s=s):
                fire_in(j + 3, (s + 3) % 4)

            pl.when((j + 3 < _NKA - 1) & (vblk(j + 3) < _NVB))(prefetch)

            def process(j=j, s=s):
                drain_in(j, s)
                pl.when(j >= 4)(lambda: drain_out(j - 4, s))
                transform(s)
                fire_out(j, s)

            pl.when((j < _NKA - 1) & (vblk(j) < _NVB))(process)
        return 0

    lax.fori_loop(0, 62, m_body, 0)
    drain_out(240, 0)
    drain_out(241, 1)
    drain_out(242, 2)
    drain_out(243, 3)


@functools.partial(
    pl.kernel,
    mesh=_mesh,
    out_type=jax.ShapeDtypeStruct((_T, 8, _NBT, 8, _LANES), jnp.float32),
    scratch_types=[
        pltpu.VMEM((_LANES, _T), jnp.int32),
        pltpu.VMEM((_NBUF, _LANES), jnp.int32),
        pltpu.VMEM((_NBUF * _LANES, _HIDDEN), jnp.float32),
        pltpu.VMEM((_NBUF, 8, 8, _LANES), jnp.float32),
        pltpu.VMEM((_LANES * 65 + 16,), jnp.float32),
        pltpu.SemaphoreType.DMA,
        pltpu.SemaphoreType.DMA,
        pltpu.SemaphoreType.DMA,
        pltpu.SemaphoreType.DMA,
        pltpu.SemaphoreType.DMA,
        pltpu.SemaphoreType.DMA,
        pltpu.SemaphoreType.DMA,
        pltpu.SemaphoreType.DMA,
        pltpu.SemaphoreType.DMA,
        pltpu.SemaphoreType.DMA,
    ],
    compiler_params=pltpu.CompilerParams(
        use_tc_tiling_on_sc=False, needs_layout_passes=False),
)
def _embed(ids_hbm, table_hbm, out_hbm, ids_v, idxb, rows_v, outb, stg,
           g0, g1, g2, g3, g4, o0, o1, o2, o3, o4):
    gsems = (g0, g1, g2, g3, g4)
    osems = (o0, o1, o2, o3, o4)
    nc = 2
    wid = lax.axis_index("s") * nc + lax.axis_index("c")

    iota = lax.iota(jnp.int32, 16)
    base8 = [iota + 16 * dg for dg in range(8)]

    def build_idx(slot, t):
        ts = jnp.full((16,), t, jnp.int32)
        for g in range(8):
            v = plsc.load_gather(ids_v, [base8[g], ts])
            idxb[slot, pl.ds(16 * g, 16)] = v

    def rows_slice(slot):
        return rows_v.at[pl.ds(slot * _LANES, _LANES)]

    def fire_gather(slot):
        pltpu.async_copy(table_hbm.at[idxb.at[slot]], rows_slice(slot),
                         gsems[slot])

    def drain_gather(slot):
        pltpu.make_async_copy(table_hbm.at[idxb.at[slot]], rows_slice(slot),
                              gsems[slot]).wait()

    _P = 65
    flat65 = [(iota + dg * 16) * _P for dg in range(8)]

    def transform(slot):
        @plsc.parallel_loop(0, _LANES, unroll=2)
        def skew_body(r):
            row = slot * _LANES + r
            sb = r * _P
            for k in range(4):
                v = rows_v[row, pl.ds(16 * k, 16)]
                plsc.store_scatter(stg, [iota + (sb + 16 * k)], v)

        @plsc.parallel_loop(0, _HIDDEN, unroll=4)
        def h_body(h):
            a = h // 8
            c = h % 8
            for dg in range(8):
                v = plsc.load_gather(stg, [flat65[dg] + h])
                outb[slot, a, c, pl.ds(dg * 16, 16)] = v * _SCALE

    def fire_write(slot, t, bt):
        pltpu.async_copy(outb.at[slot], out_hbm.at[t, :, bt], osems[slot])

    def drain_write(slot, t, bt):
        pltpu.make_async_copy(outb.at[slot], out_hbm.at[t, :, bt],
                              osems[slot]).wait()

    def bt_body(i, _):
        bt = wid * _BT_PER_W + i
        pltpu.sync_copy(ids_hbm.at[pl.ds(bt * _LANES, _LANES)], ids_v)
        for u in range(_LEAD):
            build_idx(u, u)
            fire_gather(u)

        def kk_body(kk, _):
            for s in range(_NBUF):
                u = kk * _NBUF + s
                drain_gather(s)
                pl.when(kk >= 1)(lambda s=s, u=u: drain_write(s, u, bt))
                transform(s)
                fire_write(s, u, bt)

                def refill(s=s, u=u):
                    s2 = (s + _LEAD) % _NBUF
                    build_idx(s2, u + _LEAD)
                    fire_gather(s2)

                pl.when(u + _LEAD < _T)(refill)
            return 0

        lax.fori_loop(0, _T // _NBUF, kk_body, 0)
        for s in range(_NBUF):
            drain_write(s, s, bt)
        return 0

    lax.fori_loop(0, _BT_PER_W, bt_body, 0)


def kernel(input_ids, embed_weight):
    ids = input_ids.astype(jnp.int32)
    ctab = _compact(embed_weight.T)
    out5 = _embed(ids, ctab.reshape(_VPAD, _HIDDEN))
    return jnp.transpose(out5, (2, 4, 0, 1, 3)).reshape(_B, _T, _HIDDEN)
